# speedup vs baseline: 57.9507x; 57.9507x over previous
import functools

import jax
import jax.numpy as jnp
from jax.experimental import pallas as pl


def _pick_row_block(n, cap=2048, mult=8):
    best = mult
    for d in range(mult, min(n, cap) + 1, mult):
        if n % d == 0:
            best = d
    return best


def _deg_kernel(adj_ref, deg_ref):
    k = pl.program_id(1)

    @pl.when(k == 0)
    def _init():
        deg_ref[...] = jnp.ones_like(deg_ref)

    deg_ref[...] += jnp.sum(adj_ref[...], axis=0, keepdims=True)


def _proj_kernel(h_ref, w_ref, u_ref, p_ref):
    p_ref[...] = u_ref[...] * jnp.dot(
        h_ref[...], w_ref[...], preferred_element_type=jnp.float32
    )


def _conv_kernel(adj_ref, p_ref, pself_ref, u_ref, b_ref, o_ref, *, nk, act):
    k = pl.program_id(1)

    @pl.when(k == 0)
    def _init():
        o_ref[...] = jnp.zeros_like(o_ref)

    o_ref[...] += jax.lax.dot_general(
        adj_ref[...],
        p_ref[...],
        (((0,), (0,)), ((), ())),
        preferred_element_type=jnp.float32,
    )

    @pl.when(k == nk - 1)
    def _fin():
        z = u_ref[...] * (o_ref[...] + pself_ref[...]) + b_ref[...]
        if act:
            z = jnp.maximum(z, 0.0)
        o_ref[...] = z


def kernel(x, adj, W1, b1, W2, b2, W3, b3):
    n, feat = x.shape
    hid = W1.shape[1]

    s_blk = _pick_row_block(n)
    d_blk = min(1024, ((n + 127) // 128) * 128)
    nk = n // s_blk
    ni = (n + d_blk - 1) // d_blk

    deg = pl.pallas_call(
        _deg_kernel,
        grid=(ni, nk),
        in_specs=[pl.BlockSpec((s_blk, d_blk), lambda i, k: (k, i))],
        out_specs=pl.BlockSpec((1, d_blk), lambda i, k: (0, i)),
        out_shape=jax.ShapeDtypeStruct((1, n), jnp.float32),
    )(adj)

    u = jax.lax.rsqrt(deg[0])
    U = jnp.broadcast_to(u[:, None], (n, hid))

    proj = pl.pallas_call(
        _proj_kernel,
        grid=(nk,),
        in_specs=[
            pl.BlockSpec((s_blk, feat), lambda r: (r, 0)),
            pl.BlockSpec((feat, hid), lambda r: (0, 0)),
            pl.BlockSpec((s_blk, hid), lambda r: (r, 0)),
        ],
        out_specs=pl.BlockSpec((s_blk, hid), lambda r: (r, 0)),
        out_shape=jax.ShapeDtypeStruct((n, hid), jnp.float32),
    )

    def conv(p, b, act):
        return pl.pallas_call(
            functools.partial(_conv_kernel, nk=nk, act=act),
            grid=(ni, nk),
            in_specs=[
                pl.BlockSpec((s_blk, d_blk), lambda i, k: (k, i)),
                pl.BlockSpec((s_blk, hid), lambda i, k: (k, 0)),
                pl.BlockSpec((d_blk, hid), lambda i, k: (i, 0)),
                pl.BlockSpec((d_blk, hid), lambda i, k: (i, 0)),
                pl.BlockSpec((1, hid), lambda i, k: (0, 0)),
            ],
            out_specs=pl.BlockSpec((d_blk, hid), lambda i, k: (i, 0)),
            out_shape=jax.ShapeDtypeStruct((n, hid), jnp.float32),
        )(adj, p, p, U, b)

    h = x
    for W, b, act in ((W1, b1, True), (W2, b2, True), (W3, b3, False)):
        p = proj(h, W, U)
        h = conv(p, b.reshape(1, hid), act)
    return h


# speedup vs baseline: 62.3136x; 1.0753x over previous
import functools

import jax
import jax.numpy as jnp
from jax.experimental import pallas as pl


def _pick_row_block(n, cap=2048, mult=16):
    best = mult
    for d in range(mult, min(n, cap) + 1, mult):
        if n % d == 0:
            best = d
    return best


def _deg_cast_kernel(adj_ref, deg_ref, adjc_ref):
    k = pl.program_id(1)

    @pl.when(k == 0)
    def _init():
        deg_ref[...] = jnp.ones_like(deg_ref)

    blk = adj_ref[...]
    adjc_ref[...] = blk.astype(jnp.bfloat16)
    deg_ref[...] += jnp.sum(blk, axis=0, keepdims=True)


def _proj_kernel(h_ref, w_ref, u_ref, p_ref):
    p_ref[...] = u_ref[...] * jnp.dot(
        h_ref[...], w_ref[...], preferred_element_type=jnp.float32
    )


def _conv_kernel(adjc_ref, p_ref, pself_ref, u_ref, b_ref, o_ref, *, nk, act):
    k = pl.program_id(1)

    @pl.when(k == 0)
    def _init():
        o_ref[...] = jnp.zeros_like(o_ref)

    o_ref[...] += jax.lax.dot_general(
        adjc_ref[...],
        p_ref[...].astype(jnp.bfloat16),
        (((0,), (0,)), ((), ())),
        preferred_element_type=jnp.float32,
    )

    @pl.when(k == nk - 1)
    def _fin():
        z = u_ref[...] * (o_ref[...] + pself_ref[...]) + b_ref[...]
        if act:
            z = jnp.maximum(z, 0.0)
        o_ref[...] = z


def kernel(x, adj, W1, b1, W2, b2, W3, b3):
    n, feat = x.shape
    hid = W1.shape[1]

    s_blk = _pick_row_block(n)
    d_blk = min(1024, ((n + 127) // 128) * 128)
    nk = n // s_blk
    ni = (n + d_blk - 1) // d_blk

    deg, adjc = pl.pallas_call(
        _deg_cast_kernel,
        grid=(ni, nk),
        in_specs=[pl.BlockSpec((s_blk, d_blk), lambda i, k: (k, i))],
        out_specs=[
            pl.BlockSpec((1, d_blk), lambda i, k: (0, i)),
            pl.BlockSpec((s_blk, d_blk), lambda i, k: (k, i)),
        ],
        out_shape=[
            jax.ShapeDtypeStruct((1, n), jnp.float32),
            jax.ShapeDtypeStruct((n, n), jnp.bfloat16),
        ],
    )(adj)

    u = jax.lax.rsqrt(deg[0])
    U = jnp.broadcast_to(u[:, None], (n, hid))

    proj = pl.pallas_call(
        _proj_kernel,
        grid=(nk,),
        in_specs=[
            pl.BlockSpec((s_blk, feat), lambda r: (r, 0)),
            pl.BlockSpec((feat, hid), lambda r: (0, 0)),
            pl.BlockSpec((s_blk, hid), lambda r: (r, 0)),
        ],
        out_specs=pl.BlockSpec((s_blk, hid), lambda r: (r, 0)),
        out_shape=jax.ShapeDtypeStruct((n, hid), jnp.float32),
    )

    def conv(p, b, act):
        return pl.pallas_call(
            functools.partial(_conv_kernel, nk=nk, act=act),
            grid=(ni, nk),
            in_specs=[
                pl.BlockSpec((s_blk, d_blk), lambda i, k: (k, i)),
                pl.BlockSpec((s_blk, hid), lambda i, k: (k, 0)),
                pl.BlockSpec((d_blk, hid), lambda i, k: (i, 0)),
                pl.BlockSpec((d_blk, hid), lambda i, k: (i, 0)),
                pl.BlockSpec((1, hid), lambda i, k: (0, 0)),
            ],
            out_specs=pl.BlockSpec((d_blk, hid), lambda i, k: (i, 0)),
            out_shape=jax.ShapeDtypeStruct((n, hid), jnp.float32),
        )(adjc, p, p, U, b)

    h = x
    for W, b, act in ((W1, b1, True), (W2, b2, True), (W3, b3, False)):
        p = proj(h, W, U)
        h = conv(p, b.reshape(1, hid), act)
    return h


# speedup vs baseline: 73.9515x; 1.1868x over previous
import functools

import jax
import jax.numpy as jnp
from jax.experimental import pallas as pl


def _pick_row_block(n, cap=2048, mult=16):
    best = mult
    for d in range(mult, min(n, cap) + 1, mult):
        if n % d == 0:
            best = d
    return best


def _deg_cast_kernel(adj_ref, deg_ref, adjc_ref):
    k = pl.program_id(1)

    @pl.when(k == 0)
    def _init():
        deg_ref[...] = jnp.ones_like(deg_ref)

    blk = adj_ref[...]
    adjc_ref[...] = blk.astype(jnp.float8_e4m3fn)
    deg_ref[...] += jnp.sum(blk, axis=0, keepdims=True)


def _proj_kernel(h_ref, w_ref, u_ref, p_ref):
    p_ref[...] = u_ref[...] * jnp.dot(
        h_ref[...], w_ref[...], preferred_element_type=jnp.float32
    )


def _conv_kernel(adjc_ref, p_ref, pself_ref, u_ref, b_ref, o_ref, *, nk, act):
    k = pl.program_id(1)

    @pl.when(k == 0)
    def _init():
        o_ref[...] = jnp.zeros_like(o_ref)

    o_ref[...] += jax.lax.dot_general(
        adjc_ref[...].astype(jnp.bfloat16),
        p_ref[...].astype(jnp.bfloat16),
        (((0,), (0,)), ((), ())),
        preferred_element_type=jnp.float32,
    )

    @pl.when(k == nk - 1)
    def _fin():
        z = u_ref[...] * (o_ref[...] + pself_ref[...]) + b_ref[...]
        if act:
            z = jnp.maximum(z, 0.0)
        o_ref[...] = z


def kernel(x, adj, W1, b1, W2, b2, W3, b3):
    n, feat = x.shape
    hid = W1.shape[1]

    s_blk = _pick_row_block(n)
    d_blk = min(1024, ((n + 127) // 128) * 128)
    nk = n // s_blk
    ni = (n + d_blk - 1) // d_blk

    deg, adjc = pl.pallas_call(
        _deg_cast_kernel,
        grid=(ni, nk),
        in_specs=[pl.BlockSpec((s_blk, d_blk), lambda i, k: (k, i))],
        out_specs=[
            pl.BlockSpec((1, d_blk), lambda i, k: (0, i)),
            pl.BlockSpec((s_blk, d_blk), lambda i, k: (k, i)),
        ],
        out_shape=[
            jax.ShapeDtypeStruct((1, n), jnp.float32),
            jax.ShapeDtypeStruct((n, n), jnp.float8_e4m3fn),
        ],
    )(adj)

    u = jax.lax.rsqrt(deg[0])
    U = jnp.broadcast_to(u[:, None], (n, hid))

    proj = pl.pallas_call(
        _proj_kernel,
        grid=(nk,),
        in_specs=[
            pl.BlockSpec((s_blk, feat), lambda r: (r, 0)),
            pl.BlockSpec((feat, hid), lambda r: (0, 0)),
            pl.BlockSpec((s_blk, hid), lambda r: (r, 0)),
        ],
        out_specs=pl.BlockSpec((s_blk, hid), lambda r: (r, 0)),
        out_shape=jax.ShapeDtypeStruct((n, hid), jnp.float32),
    )

    def conv(p, b, act):
        return pl.pallas_call(
            functools.partial(_conv_kernel, nk=nk, act=act),
            grid=(ni, nk),
            in_specs=[
                pl.BlockSpec((s_blk, d_blk), lambda i, k: (k, i)),
                pl.BlockSpec((s_blk, hid), lambda i, k: (k, 0)),
                pl.BlockSpec((d_blk, hid), lambda i, k: (i, 0)),
                pl.BlockSpec((d_blk, hid), lambda i, k: (i, 0)),
                pl.BlockSpec((1, hid), lambda i, k: (0, 0)),
            ],
            out_specs=pl.BlockSpec((d_blk, hid), lambda i, k: (i, 0)),
            out_shape=jax.ShapeDtypeStruct((n, hid), jnp.float32),
        )(adjc, p, p, U, b)

    h = x
    for W, b, act in ((W1, b1, True), (W2, b2, True), (W3, b3, False)):
        p = proj(h, W, U)
        h = conv(p, b.reshape(1, hid), act)
    return h
